# baseline (speedup 1.0000x reference)
import functools

import jax
import jax.numpy as jnp
from jax import lax
from jax.experimental import pallas as pl
from jax.experimental.pallas import tpu as pltpu
from jax.experimental.pallas import tpu_sc as plsc

NC = 2
NS = 16
NW = NC * NS
CH = 128
LANES = 16


def _round_up(a, b):
  return (a + b - 1) // b * b


def _sc_deg(src_dst_len, n_pad, dst_pad):
  e_pad = src_dst_len
  per_w = e_pad // NW
  n_chunks = per_w // CH
  rows_per_tile = n_pad // NS
  mesh = plsc.VectorSubcoreMesh(core_axis_name="c", subcore_axis_name="s")

  @functools.partial(
      pl.kernel,
      out_type=jax.ShapeDtypeStruct((NC, n_pad), jnp.float32),
      mesh=mesh,
      scratch_types=[
          pltpu.VMEM((CH,), jnp.int32),
          pltpu.VMEM((CH,), jnp.float32),
          pltpu.VMEM((CH,), jnp.float32),
          pltpu.VMEM_SHARED((n_pad,), jnp.float32),
      ],
  )
  def k(dst_hbm, out_hbm, dv, ones_v, zbuf, deg_sh):
    c = lax.axis_index("c")
    s = lax.axis_index("s")
    wid = c * NS + s
    one16 = jnp.full((LANES,), 1.0, jnp.float32)
    z16 = jnp.zeros((LANES,), jnp.float32)
    for i in range(CH // LANES):
      ones_v[pl.ds(i * LANES, LANES)] = one16
      zbuf[pl.ds(i * LANES, LANES)] = z16
    tile_base = s * rows_per_tile
    for i in range(rows_per_tile // CH):
      pltpu.sync_copy(zbuf, deg_sh.at[pl.ds(tile_base + i * CH, CH)])
    plsc.subcore_barrier()

    base = wid * per_w

    def chunk(j, _):
      off = base + j * CH
      pltpu.sync_copy(dst_hbm.at[pl.ds(off, CH)], dv)
      pltpu.sync_copy(ones_v, deg_sh.at[dv], add=True)
      return 0

    lax.fori_loop(0, n_chunks, chunk, 0)
    plsc.subcore_barrier()
    for i in range(rows_per_tile // CH):
      r0 = tile_base + i * CH
      pltpu.sync_copy(deg_sh.at[pl.ds(r0, CH)], zbuf)
      pltpu.sync_copy(zbuf, out_hbm.at[c, pl.ds(r0, CH)])

  return k(dst_pad)


def _sc_aggregate(e_pad, n_pad, h, k0, src_pad, dst_pad, table):
  total_chunks = e_pad // CH
  k1 = total_chunks // NS - k0
  rows_per_tile = n_pad // NS
  mesh = plsc.VectorSubcoreMesh(core_axis_name="c", subcore_axis_name="s")

  @functools.partial(
      pl.kernel,
      out_type=jax.ShapeDtypeStruct((NC, n_pad, h), jnp.float32),
      mesh=mesh,
      scratch_types=[
          pltpu.VMEM((CH,), jnp.int32),
          pltpu.VMEM((CH,), jnp.int32),
          pltpu.VMEM((CH, h), jnp.float32),
          pltpu.VMEM_SHARED((n_pad, h), jnp.float32),
          pltpu.SemaphoreType.DMA,
      ],
  )
  def k(src_hbm, dst_hbm, table_hbm, out_hbm, sv, dv, buf, acc_sh, gsem):
    c = lax.axis_index("c")
    s = lax.axis_index("s")
    wid = c * NS + s
    z16 = jnp.zeros((LANES,), jnp.float32)

    def zrow(r, _):
      for cc in range(h // LANES):
        buf[r, pl.ds(cc * LANES, LANES)] = z16
      return 0

    lax.fori_loop(0, CH, zrow, 0)
    tile_base = s * rows_per_tile
    for i in range(rows_per_tile // CH):
      pltpu.sync_copy(buf, acc_sh.at[pl.ds(tile_base + i * CH, CH)])
    plsc.subcore_barrier()

    my_chunks = jnp.where(c == 0, k0, k1)
    base_chunk = jnp.where(c == 0, s * k0, NS * k0 + s * k1)

    def chunk(j, _):
      off = (base_chunk + j) * CH
      pltpu.sync_copy(src_hbm.at[pl.ds(off, CH)], sv)
      pltpu.sync_copy(dst_hbm.at[pl.ds(off, CH)], dv)
      pltpu.async_copy(table_hbm.at[sv], buf, gsem).wait()
      pltpu.sync_copy(buf, acc_sh.at[dv], add=True)
      return 0

    lax.fori_loop(0, my_chunks, chunk, 0)
    plsc.subcore_barrier()
    for i in range(rows_per_tile // CH):
      r0 = tile_base + i * CH
      pltpu.sync_copy(acc_sh.at[pl.ds(r0, CH)], buf)
      pltpu.sync_copy(buf, out_hbm.at[c, pl.ds(r0, CH)])

  return k(src_pad, dst_pad, table)


def _sc_pairgather(e_pad, h, k0, src_pad, dst_pad, p_tab, q_tab):
  total_chunks = e_pad // CH
  k1 = total_chunks // NS - k0
  mesh = plsc.VectorSubcoreMesh(core_axis_name="c", subcore_axis_name="s")

  @functools.partial(
      pl.kernel,
      out_type=jax.ShapeDtypeStruct((e_pad, h), jnp.float32),
      mesh=mesh,
      scratch_types=[
          pltpu.VMEM((CH,), jnp.int32),
          pltpu.VMEM((CH,), jnp.int32),
          pltpu.VMEM((CH, h), jnp.float32),
          pltpu.VMEM((CH, h), jnp.float32),
          pltpu.SemaphoreType.DMA,
          pltpu.SemaphoreType.DMA,
      ],
  )
  def k(src_hbm, dst_hbm, p_hbm, q_hbm, out_hbm, sv, dv, bp, bq, sp, sq):
    c = lax.axis_index("c")
    s = lax.axis_index("s")
    my_chunks = jnp.where(c == 0, k0, k1)
    base_chunk = jnp.where(c == 0, s * k0, NS * k0 + s * k1)

    def chunk(j, _):
      off = (base_chunk + j) * CH
      pltpu.sync_copy(src_hbm.at[pl.ds(off, CH)], sv)
      pltpu.sync_copy(dst_hbm.at[pl.ds(off, CH)], dv)
      cp = pltpu.async_copy(p_hbm.at[sv], bp, sp)
      cq = pltpu.async_copy(q_hbm.at[dv], bq, sq)
      cp.wait()
      cq.wait()

      def addrow(r, _):
        for cc in range(h // LANES):
          sl = pl.ds(cc * LANES, LANES)
          bp[r, sl] = bp[r, sl] + bq[r, sl]
        return 0

      lax.fori_loop(0, CH, addrow, 0)
      pltpu.sync_copy(bp, out_hbm.at[pl.ds(off, CH)])
      return 0

    lax.fori_loop(0, my_chunks, chunk, 0)

  return k(src_pad, dst_pad, p_tab, q_tab)


def _tc_layer1(x_pad, w1, deg_t, bn):
  n_pad, d = x_pad.shape
  h = w1.shape[1]

  def body(x_ref, w_ref, deg_ref, g_ref, dinv_ref):
    deg = deg_ref[:, 0:1] + deg_ref[:, 1:2] + 1.0
    dinv = lax.rsqrt(deg)
    dinv_ref[...] = dinv
    g_ref[...] = jnp.dot(x_ref[...], w_ref[...],
                         preferred_element_type=jnp.float32) * dinv

  return pl.pallas_call(
      body,
      grid=(n_pad // bn,),
      in_specs=[
          pl.BlockSpec((bn, d), lambda i: (i, 0)),
          pl.BlockSpec((d, h), lambda i: (0, 0)),
          pl.BlockSpec((bn, 2), lambda i: (i, 0)),
      ],
      out_specs=[
          pl.BlockSpec((bn, h), lambda i: (i, 0)),
          pl.BlockSpec((bn, 1), lambda i: (i, 0)),
      ],
      out_shape=[
          jax.ShapeDtypeStruct((n_pad, h), jnp.float32),
          jax.ShapeDtypeStruct((n_pad, 1), jnp.float32),
      ],
  )(x_pad, w1, deg_t)


def _tc_layer2(g1, agg1, dinv, b1, w2, bn):
  n_pad, h = g1.shape
  d2 = w2.shape[1]

  def body(g_ref, agg_ref, dinv_ref, b_ref, w_ref, out_ref):
    dinv = dinv_ref[...]
    pre = (agg_ref[0] + agg_ref[1] + g_ref[...]) * dinv + b_ref[...]
    h1 = jnp.maximum(pre, 0.0)
    out_ref[...] = jnp.dot(h1, w_ref[...],
                           preferred_element_type=jnp.float32) * dinv

  return pl.pallas_call(
      body,
      grid=(n_pad // bn,),
      in_specs=[
          pl.BlockSpec((bn, h), lambda i: (i, 0)),
          pl.BlockSpec((NC, bn, h), lambda i: (0, i, 0)),
          pl.BlockSpec((bn, 1), lambda i: (i, 0)),
          pl.BlockSpec((1, h), lambda i: (0, 0)),
          pl.BlockSpec((h, d2), lambda i: (0, 0)),
      ],
      out_specs=pl.BlockSpec((bn, d2), lambda i: (i, 0)),
      out_shape=jax.ShapeDtypeStruct((n_pad, d2), jnp.float32),
  )(g1, agg1, dinv, b1, w2)


def _tc_node_mlp(g2, agg2, dinv, b2, wo1, bo1, wo2, bo2, we1a, we1b, be1, bn):
  n_pad, d = g2.shape
  h = wo1.shape[1]

  def body(g_ref, agg_ref, dinv_ref, b2_ref, wo1_ref, bo1_ref, wo2_ref,
           bo2_ref, wa_ref, wb_ref, be1_ref, h_ref, p_ref, q_ref):
    dinv = dinv_ref[...]
    pre = (agg_ref[0] + agg_ref[1] + g_ref[...]) * dinv + b2_ref[...]
    h2 = jnp.maximum(pre, 0.0)
    t = jnp.maximum(
        jnp.dot(h2, wo1_ref[...], preferred_element_type=jnp.float32)
        + bo1_ref[...], 0.0)
    hout = jnp.dot(t, wo2_ref[...],
                   preferred_element_type=jnp.float32) + bo2_ref[...]
    h_ref[...] = hout
    p_ref[...] = jnp.dot(hout, wa_ref[...],
                         preferred_element_type=jnp.float32) + be1_ref[...]
    q_ref[...] = jnp.dot(hout, wb_ref[...],
                         preferred_element_type=jnp.float32)

  return pl.pallas_call(
      body,
      grid=(n_pad // bn,),
      in_specs=[
          pl.BlockSpec((bn, d), lambda i: (i, 0)),
          pl.BlockSpec((NC, bn, d), lambda i: (0, i, 0)),
          pl.BlockSpec((bn, 1), lambda i: (i, 0)),
          pl.BlockSpec((1, d), lambda i: (0, 0)),
          pl.BlockSpec((d, h), lambda i: (0, 0)),
          pl.BlockSpec((1, h), lambda i: (0, 0)),
          pl.BlockSpec((h, d), lambda i: (0, 0)),
          pl.BlockSpec((1, d), lambda i: (0, 0)),
          pl.BlockSpec((d, h), lambda i: (0, 0)),
          pl.BlockSpec((d, h), lambda i: (0, 0)),
          pl.BlockSpec((1, h), lambda i: (0, 0)),
      ],
      out_specs=[
          pl.BlockSpec((bn, d), lambda i: (i, 0)),
          pl.BlockSpec((bn, h), lambda i: (i, 0)),
          pl.BlockSpec((bn, h), lambda i: (i, 0)),
      ],
      out_shape=[
          jax.ShapeDtypeStruct((n_pad, d), jnp.float32),
          jax.ShapeDtypeStruct((n_pad, h), jnp.float32),
          jax.ShapeDtypeStruct((n_pad, h), jnp.float32),
      ],
  )(g2, agg2, dinv, b2, wo1, bo1, wo2, bo2, we1a, we1b, be1)


def _tc_edge_mlp(s_arr, edge_attr, we1c, we2, be2, be):
  e_num, de = edge_attr.shape
  h = s_arr.shape[1]

  def body(s_ref, ea_ref, wc_ref, w2_ref, b2_ref, e_ref):
    t = jnp.maximum(
        s_ref[...] + jnp.dot(ea_ref[...], wc_ref[...],
                             preferred_element_type=jnp.float32), 0.0)
    e_ref[...] = jnp.dot(t, w2_ref[...],
                         preferred_element_type=jnp.float32) + b2_ref[...]

  return pl.pallas_call(
      body,
      grid=(e_num // be,),
      in_specs=[
          pl.BlockSpec((be, h), lambda i: (i, 0)),
          pl.BlockSpec((be, de), lambda i: (i, 0)),
          pl.BlockSpec((de, h), lambda i: (0, 0)),
          pl.BlockSpec((h, de), lambda i: (0, 0)),
          pl.BlockSpec((1, de), lambda i: (0, 0)),
      ],
      out_specs=pl.BlockSpec((be, de), lambda i: (i, 0)),
      out_shape=jax.ShapeDtypeStruct((e_num, de), jnp.float32),
  )(s_arr, edge_attr, we1c, we2, be2)


def kernel(x, edge_index, edge_attr, W1, b1, W2, b2, Wo1, bo1, Wo2, bo2,
           We1, be1, We2, be2):
  n, d = x.shape
  h = W1.shape[1]
  e_num = edge_index.shape[1]

  e_pad = _round_up(e_num, NW * CH)
  n_pad = _round_up(n + 1, NS * CH)

  src = edge_index[0]
  dst = edge_index[1]
  pad_idx = jnp.full((e_pad - e_num,), n, jnp.int32)
  src_pad = jnp.concatenate([src, pad_idx])
  dst_pad = jnp.concatenate([dst, pad_idx])
  x_pad = jnp.zeros((n_pad, d), jnp.float32).at[:n].set(x)

  bn = 1024
  assert n_pad % bn == 0

  degp = _sc_deg(e_pad, n_pad, dst_pad)
  deg_t = degp.T

  g1, dinv = _tc_layer1(x_pad, W1, deg_t, bn)
  total_per_sub = e_pad // CH // NS
  k0 = (total_per_sub * 7) // 10
  agg1 = _sc_aggregate(e_pad, n_pad, h, k0, src_pad, dst_pad, g1)
  g2 = _tc_layer2(g1, agg1, dinv, b1.reshape(1, -1), W2, bn)
  agg2 = _sc_aggregate(e_pad, n_pad, d, k0, src_pad, dst_pad, g2)
  hout, p_tab, q_tab = _tc_node_mlp(
      g2, agg2, dinv, b2.reshape(1, -1), Wo1, bo1.reshape(1, -1), Wo2,
      bo2.reshape(1, -1), We1[:d], We1[d:2 * d], be1.reshape(1, -1), bn)

  s_arr = _sc_pairgather(e_pad, h, k0, src_pad, dst_pad, p_tab, q_tab)
  e_out = _tc_edge_mlp(s_arr, edge_attr, We1[2 * d:], We2,
                       be2.reshape(1, -1), 2000)

  return hout[:n], e_out


# speedup vs baseline: 1.0309x; 1.0309x over previous
import functools

import jax
import jax.numpy as jnp
from jax import lax
from jax.experimental import pallas as pl
from jax.experimental.pallas import tpu as pltpu
from jax.experimental.pallas import tpu_sc as plsc

NC = 2
NS = 16
NW = NC * NS
CH = 128
LANES = 16


def _round_up(a, b):
  return (a + b - 1) // b * b


def _sc_deg(src_dst_len, n_pad, dst_pad):
  e_pad = src_dst_len
  per_w = e_pad // NW
  n_chunks = per_w // CH
  rows_per_tile = n_pad // NS
  mesh = plsc.VectorSubcoreMesh(core_axis_name="c", subcore_axis_name="s")

  @functools.partial(
      pl.kernel,
      out_type=jax.ShapeDtypeStruct((NC, n_pad), jnp.float32),
      mesh=mesh,
      scratch_types=[
          pltpu.VMEM((CH,), jnp.int32),
          pltpu.VMEM((CH,), jnp.float32),
          pltpu.VMEM((CH,), jnp.float32),
          pltpu.VMEM_SHARED((n_pad,), jnp.float32),
      ],
  )
  def k(dst_hbm, out_hbm, dv, ones_v, zbuf, deg_sh):
    c = lax.axis_index("c")
    s = lax.axis_index("s")
    wid = c * NS + s
    one16 = jnp.full((LANES,), 1.0, jnp.float32)
    z16 = jnp.zeros((LANES,), jnp.float32)
    for i in range(CH // LANES):
      ones_v[pl.ds(i * LANES, LANES)] = one16
      zbuf[pl.ds(i * LANES, LANES)] = z16
    tile_base = s * rows_per_tile
    for i in range(rows_per_tile // CH):
      pltpu.sync_copy(zbuf, deg_sh.at[pl.ds(tile_base + i * CH, CH)])
    plsc.subcore_barrier()

    base = wid * per_w

    def chunk(j, _):
      off = base + j * CH
      pltpu.sync_copy(dst_hbm.at[pl.ds(off, CH)], dv)
      pltpu.sync_copy(ones_v, deg_sh.at[dv], add=True)
      return 0

    lax.fori_loop(0, n_chunks, chunk, 0)
    plsc.subcore_barrier()
    for i in range(rows_per_tile // CH):
      r0 = tile_base + i * CH
      pltpu.sync_copy(deg_sh.at[pl.ds(r0, CH)], zbuf)
      pltpu.sync_copy(zbuf, out_hbm.at[c, pl.ds(r0, CH)])

  return k(dst_pad)


def _sc_aggregate(e_pad, n_pad, h, k0, src_pad, dst_pad, table):
  total_chunks = e_pad // CH
  k1 = total_chunks // NS - k0
  rows_per_tile = n_pad // NS
  mesh = plsc.VectorSubcoreMesh(core_axis_name="c", subcore_axis_name="s")

  @functools.partial(
      pl.kernel,
      out_type=jax.ShapeDtypeStruct((NC, n_pad, h), jnp.float32),
      mesh=mesh,
      scratch_types=[
          pltpu.VMEM((CH,), jnp.int32),
          pltpu.VMEM((CH,), jnp.int32),
          pltpu.VMEM((CH, h), jnp.float32),
          pltpu.VMEM_SHARED((n_pad, h), jnp.float32),
          pltpu.SemaphoreType.DMA,
      ],
  )
  def k(src_hbm, dst_hbm, table_hbm, out_hbm, sv, dv, buf, acc_sh, gsem):
    c = lax.axis_index("c")
    s = lax.axis_index("s")
    wid = c * NS + s
    z16 = jnp.zeros((LANES,), jnp.float32)

    def zrow(r, _):
      for cc in range(h // LANES):
        buf[r, pl.ds(cc * LANES, LANES)] = z16
      return 0

    lax.fori_loop(0, CH, zrow, 0)
    tile_base = s * rows_per_tile
    for i in range(rows_per_tile // CH):
      pltpu.sync_copy(buf, acc_sh.at[pl.ds(tile_base + i * CH, CH)])
    plsc.subcore_barrier()

    my_chunks = jnp.where(c == 0, k0, k1)
    base_chunk = jnp.where(c == 0, s * k0, NS * k0 + s * k1)

    def chunk(j, _):
      off = (base_chunk + j) * CH
      pltpu.sync_copy(src_hbm.at[pl.ds(off, CH)], sv)
      pltpu.sync_copy(dst_hbm.at[pl.ds(off, CH)], dv)
      pltpu.async_copy(table_hbm.at[sv], buf, gsem).wait()
      pltpu.sync_copy(buf, acc_sh.at[dv], add=True)
      return 0

    lax.fori_loop(0, my_chunks, chunk, 0)
    plsc.subcore_barrier()
    for i in range(rows_per_tile // CH):
      r0 = tile_base + i * CH
      pltpu.sync_copy(acc_sh.at[pl.ds(r0, CH)], buf)
      pltpu.sync_copy(buf, out_hbm.at[c, pl.ds(r0, CH)])

  return k(src_pad, dst_pad, table)


def _sc_pairgather(e_pad, h, k0, src_pad, dst_pad, p_tab, q_tab):
  total_chunks = e_pad // CH
  k1 = total_chunks // NS - k0
  mesh = plsc.VectorSubcoreMesh(core_axis_name="c", subcore_axis_name="s")

  @functools.partial(
      pl.kernel,
      out_type=jax.ShapeDtypeStruct((e_pad, h), jnp.float32),
      mesh=mesh,
      scratch_types=[
          pltpu.VMEM((CH,), jnp.int32),
          pltpu.VMEM((CH,), jnp.int32),
          pltpu.VMEM((CH, h), jnp.float32),
          pltpu.VMEM((CH, h), jnp.float32),
          pltpu.SemaphoreType.DMA,
          pltpu.SemaphoreType.DMA,
      ],
  )
  def k(src_hbm, dst_hbm, p_hbm, q_hbm, out_hbm, sv, dv, bp, bq, sp, sq):
    c = lax.axis_index("c")
    s = lax.axis_index("s")
    my_chunks = jnp.where(c == 0, k0, k1)
    base_chunk = jnp.where(c == 0, s * k0, NS * k0 + s * k1)

    def chunk(j, _):
      off = (base_chunk + j) * CH
      pltpu.sync_copy(src_hbm.at[pl.ds(off, CH)], sv)
      pltpu.sync_copy(dst_hbm.at[pl.ds(off, CH)], dv)
      cp = pltpu.async_copy(p_hbm.at[sv], bp, sp)
      cq = pltpu.async_copy(q_hbm.at[dv], bq, sq)
      cp.wait()
      cq.wait()

      def addrow(r, _):
        for cc in range(h // LANES):
          sl = pl.ds(cc * LANES, LANES)
          bp[r, sl] = bp[r, sl] + bq[r, sl]
        return 0

      lax.fori_loop(0, CH, addrow, 0)
      pltpu.sync_copy(bp, out_hbm.at[pl.ds(off, CH)])
      return 0

    lax.fori_loop(0, my_chunks, chunk, 0)

  return k(src_pad, dst_pad, p_tab, q_tab)


def _tc_layer1(x_pad, w1, deg_t, bn):
  n_pad, d = x_pad.shape
  h = w1.shape[1]

  def body(x_ref, w_ref, deg_ref, g_ref, dinv_ref):
    deg = deg_ref[:, 0:1] + deg_ref[:, 1:2] + 1.0
    dinv = lax.rsqrt(deg)
    dinv_ref[...] = dinv
    g_ref[...] = jnp.dot(x_ref[...], w_ref[...],
                         preferred_element_type=jnp.float32) * dinv

  return pl.pallas_call(
      body,
      grid=(n_pad // bn,),
      in_specs=[
          pl.BlockSpec((bn, d), lambda i: (i, 0)),
          pl.BlockSpec((d, h), lambda i: (0, 0)),
          pl.BlockSpec((bn, 2), lambda i: (i, 0)),
      ],
      out_specs=[
          pl.BlockSpec((bn, h), lambda i: (i, 0)),
          pl.BlockSpec((bn, 1), lambda i: (i, 0)),
      ],
      out_shape=[
          jax.ShapeDtypeStruct((n_pad, h), jnp.float32),
          jax.ShapeDtypeStruct((n_pad, 1), jnp.float32),
      ],
  )(x_pad, w1, deg_t)


def _tc_layer2(g1, agg1, dinv, b1, w2, bn):
  n_pad, h = g1.shape
  d2 = w2.shape[1]

  def body(g_ref, agg_ref, dinv_ref, b_ref, w_ref, out_ref):
    dinv = dinv_ref[...]
    pre = (agg_ref[0] + agg_ref[1] + g_ref[...]) * dinv + b_ref[...]
    h1 = jnp.maximum(pre, 0.0)
    out_ref[...] = jnp.dot(h1, w_ref[...],
                           preferred_element_type=jnp.float32) * dinv

  return pl.pallas_call(
      body,
      grid=(n_pad // bn,),
      in_specs=[
          pl.BlockSpec((bn, h), lambda i: (i, 0)),
          pl.BlockSpec((NC, bn, h), lambda i: (0, i, 0)),
          pl.BlockSpec((bn, 1), lambda i: (i, 0)),
          pl.BlockSpec((1, h), lambda i: (0, 0)),
          pl.BlockSpec((h, d2), lambda i: (0, 0)),
      ],
      out_specs=pl.BlockSpec((bn, d2), lambda i: (i, 0)),
      out_shape=jax.ShapeDtypeStruct((n_pad, d2), jnp.float32),
  )(g1, agg1, dinv, b1, w2)


def _tc_node_mlp(g2, agg2, dinv, b2, wo1, bo1, wo2, bo2, we1a, we1b, be1, bn):
  n_pad, d = g2.shape
  h = wo1.shape[1]

  def body(g_ref, agg_ref, dinv_ref, b2_ref, wo1_ref, bo1_ref, wo2_ref,
           bo2_ref, wa_ref, wb_ref, be1_ref, h_ref, p_ref, q_ref):
    dinv = dinv_ref[...]
    pre = (agg_ref[0] + agg_ref[1] + g_ref[...]) * dinv + b2_ref[...]
    h2 = jnp.maximum(pre, 0.0)
    t = jnp.maximum(
        jnp.dot(h2, wo1_ref[...], preferred_element_type=jnp.float32)
        + bo1_ref[...], 0.0)
    hout = jnp.dot(t, wo2_ref[...],
                   preferred_element_type=jnp.float32) + bo2_ref[...]
    h_ref[...] = hout
    p_ref[...] = jnp.dot(hout, wa_ref[...],
                         preferred_element_type=jnp.float32) + be1_ref[...]
    q_ref[...] = jnp.dot(hout, wb_ref[...],
                         preferred_element_type=jnp.float32)

  return pl.pallas_call(
      body,
      grid=(n_pad // bn,),
      in_specs=[
          pl.BlockSpec((bn, d), lambda i: (i, 0)),
          pl.BlockSpec((NC, bn, d), lambda i: (0, i, 0)),
          pl.BlockSpec((bn, 1), lambda i: (i, 0)),
          pl.BlockSpec((1, d), lambda i: (0, 0)),
          pl.BlockSpec((d, h), lambda i: (0, 0)),
          pl.BlockSpec((1, h), lambda i: (0, 0)),
          pl.BlockSpec((h, d), lambda i: (0, 0)),
          pl.BlockSpec((1, d), lambda i: (0, 0)),
          pl.BlockSpec((d, h), lambda i: (0, 0)),
          pl.BlockSpec((d, h), lambda i: (0, 0)),
          pl.BlockSpec((1, h), lambda i: (0, 0)),
      ],
      out_specs=[
          pl.BlockSpec((bn, d), lambda i: (i, 0)),
          pl.BlockSpec((bn, h), lambda i: (i, 0)),
          pl.BlockSpec((bn, h), lambda i: (i, 0)),
      ],
      out_shape=[
          jax.ShapeDtypeStruct((n_pad, d), jnp.float32),
          jax.ShapeDtypeStruct((n_pad, h), jnp.float32),
          jax.ShapeDtypeStruct((n_pad, h), jnp.float32),
      ],
  )(g2, agg2, dinv, b2, wo1, bo1, wo2, bo2, we1a, we1b, be1)


def _tc_edge_mlp(s_arr, edge_attr, we1c, we2, be2, be):
  e_num, de = edge_attr.shape
  h = s_arr.shape[1]

  def body(s_ref, ea_ref, wc_ref, w2_ref, b2_ref, e_ref):
    t = jnp.maximum(
        s_ref[...] + jnp.dot(ea_ref[...], wc_ref[...],
                             preferred_element_type=jnp.float32), 0.0)
    e_ref[...] = jnp.dot(t, w2_ref[...],
                         preferred_element_type=jnp.float32) + b2_ref[...]

  return pl.pallas_call(
      body,
      grid=(e_num // be,),
      in_specs=[
          pl.BlockSpec((be, h), lambda i: (i, 0)),
          pl.BlockSpec((be, de), lambda i: (i, 0)),
          pl.BlockSpec((de, h), lambda i: (0, 0)),
          pl.BlockSpec((h, de), lambda i: (0, 0)),
          pl.BlockSpec((1, de), lambda i: (0, 0)),
      ],
      out_specs=pl.BlockSpec((be, de), lambda i: (i, 0)),
      out_shape=jax.ShapeDtypeStruct((e_num, de), jnp.float32),
  )(s_arr, edge_attr, we1c, we2, be2)


def kernel(x, edge_index, edge_attr, W1, b1, W2, b2, Wo1, bo1, Wo2, bo2,
           We1, be1, We2, be2):
  n, d = x.shape
  h = W1.shape[1]
  e_num = edge_index.shape[1]

  e_pad = _round_up(e_num, NW * CH)
  n_pad = _round_up(n + 1, NS * CH)

  src = edge_index[0]
  dst = edge_index[1]
  pad_idx = jnp.full((e_pad - e_num,), n, jnp.int32)
  src_pad = jnp.concatenate([src, pad_idx])
  dst_pad = jnp.concatenate([dst, pad_idx])
  x_pad = jnp.zeros((n_pad, d), jnp.float32).at[:n].set(x)

  bn = 1024
  assert n_pad % bn == 0

  degp = _sc_deg(e_pad, n_pad, dst_pad)
  deg_t = degp.T

  g1, dinv = _tc_layer1(x_pad, W1, deg_t, bn)
  total_per_sub = e_pad // CH // NS
  k0 = (total_per_sub * 2) // 3
  agg1 = _sc_aggregate(e_pad, n_pad, h, k0, src_pad, dst_pad, g1)
  g2 = _tc_layer2(g1, agg1, dinv, b1.reshape(1, -1), W2, bn)
  agg2 = _sc_aggregate(e_pad, n_pad, d, k0, src_pad, dst_pad, g2)
  hout, p_tab, q_tab = _tc_node_mlp(
      g2, agg2, dinv, b2.reshape(1, -1), Wo1, bo1.reshape(1, -1), Wo2,
      bo2.reshape(1, -1), We1[:d], We1[d:2 * d], be1.reshape(1, -1), bn)

  s_arr = _sc_pairgather(e_pad, h, k0, src_pad, dst_pad, p_tab, q_tab)
  e_out = _tc_edge_mlp(s_arr, edge_attr, We1[2 * d:], We2,
                       be2.reshape(1, -1), 2000)

  return hout[:n], e_out


# speedup vs baseline: 1.0487x; 1.0173x over previous
import functools

import jax
import jax.numpy as jnp
from jax import lax
from jax.experimental import pallas as pl
from jax.experimental.pallas import tpu as pltpu
from jax.experimental.pallas import tpu_sc as plsc

NC = 2
NS = 16
NW = NC * NS
CH = 128
LANES = 16


def _round_up(a, b):
  return (a + b - 1) // b * b


def _sc_deg(src_dst_len, n_pad, dst_pad):
  e_pad = src_dst_len
  per_w = e_pad // NW
  n_chunks = per_w // CH
  rows_per_tile = n_pad // NS
  mesh = plsc.VectorSubcoreMesh(core_axis_name="c", subcore_axis_name="s")

  @functools.partial(
      pl.kernel,
      out_type=jax.ShapeDtypeStruct((NC, n_pad), jnp.float32),
      mesh=mesh,
      scratch_types=[
          pltpu.VMEM((CH,), jnp.int32),
          pltpu.VMEM((CH,), jnp.float32),
          pltpu.VMEM((CH,), jnp.float32),
          pltpu.VMEM_SHARED((n_pad,), jnp.float32),
      ],
  )
  def k(dst_hbm, out_hbm, dv, ones_v, zbuf, deg_sh):
    c = lax.axis_index("c")
    s = lax.axis_index("s")
    wid = c * NS + s
    one16 = jnp.full((LANES,), 1.0, jnp.float32)
    z16 = jnp.zeros((LANES,), jnp.float32)
    for i in range(CH // LANES):
      ones_v[pl.ds(i * LANES, LANES)] = one16
      zbuf[pl.ds(i * LANES, LANES)] = z16
    tile_base = s * rows_per_tile
    for i in range(rows_per_tile // CH):
      pltpu.sync_copy(zbuf, deg_sh.at[pl.ds(tile_base + i * CH, CH)])
    plsc.subcore_barrier()

    base = wid * per_w

    def chunk(j, _):
      off = base + j * CH
      pltpu.sync_copy(dst_hbm.at[pl.ds(off, CH)], dv)
      pltpu.sync_copy(ones_v, deg_sh.at[dv], add=True)
      return 0

    lax.fori_loop(0, n_chunks, chunk, 0)
    plsc.subcore_barrier()
    for i in range(rows_per_tile // CH):
      r0 = tile_base + i * CH
      pltpu.sync_copy(deg_sh.at[pl.ds(r0, CH)], zbuf)
      pltpu.sync_copy(zbuf, out_hbm.at[c, pl.ds(r0, CH)])

  return k(dst_pad)


def _sc_aggregate(e_pad, n_pad, h, k0, src_pad, dst_pad, table):
  total_chunks = e_pad // CH
  k1 = total_chunks // NS - k0
  rows_per_tile = n_pad // NS
  mesh = plsc.VectorSubcoreMesh(core_axis_name="c", subcore_axis_name="s")

  @functools.partial(
      pl.kernel,
      out_type=jax.ShapeDtypeStruct((NC, n_pad, h), jnp.float32),
      mesh=mesh,
      scratch_types=[
          pltpu.VMEM((CH,), jnp.int32),
          pltpu.VMEM((CH,), jnp.int32),
          pltpu.VMEM((CH, h), jnp.float32),
          pltpu.VMEM_SHARED((n_pad, h), jnp.float32),
          pltpu.SemaphoreType.DMA,
      ],
  )
  def k(src_hbm, dst_hbm, table_hbm, out_hbm, sv, dv, buf, acc_sh, gsem):
    c = lax.axis_index("c")
    s = lax.axis_index("s")
    wid = c * NS + s
    z16 = jnp.zeros((LANES,), jnp.float32)

    def zrow(r, _):
      for cc in range(h // LANES):
        buf[r, pl.ds(cc * LANES, LANES)] = z16
      return 0

    lax.fori_loop(0, CH, zrow, 0)
    tile_base = s * rows_per_tile
    for i in range(rows_per_tile // CH):
      pltpu.sync_copy(buf, acc_sh.at[pl.ds(tile_base + i * CH, CH)])
    plsc.subcore_barrier()

    my_chunks = jnp.where(c == 0, k0, k1)
    base_chunk = jnp.where(c == 0, s * k0, NS * k0 + s * k1)

    def chunk(j, _):
      off = (base_chunk + j) * CH
      pltpu.sync_copy(src_hbm.at[pl.ds(off, CH)], sv)
      pltpu.sync_copy(dst_hbm.at[pl.ds(off, CH)], dv)
      pltpu.async_copy(table_hbm.at[sv], buf, gsem).wait()
      pltpu.sync_copy(buf, acc_sh.at[dv], add=True)
      return 0

    lax.fori_loop(0, my_chunks, chunk, 0)
    plsc.subcore_barrier()
    for i in range(rows_per_tile // CH):
      r0 = tile_base + i * CH
      pltpu.sync_copy(acc_sh.at[pl.ds(r0, CH)], buf)
      pltpu.sync_copy(buf, out_hbm.at[c, pl.ds(r0, CH)])

  return k(src_pad, dst_pad, table)


def _sc_pairgather(e_pad, h, k0, src_pad, dst_pad, p_tab, q_tab):
  total_chunks = e_pad // CH
  k1 = total_chunks // NS - k0
  mesh = plsc.VectorSubcoreMesh(core_axis_name="c", subcore_axis_name="s")

  @functools.partial(
      pl.kernel,
      out_type=jax.ShapeDtypeStruct((e_pad, h), jnp.float32),
      mesh=mesh,
      scratch_types=[
          pltpu.VMEM((CH,), jnp.int32),
          pltpu.VMEM((CH,), jnp.int32),
          pltpu.VMEM((CH, h), jnp.float32),
          pltpu.VMEM((CH, h), jnp.float32),
          pltpu.SemaphoreType.DMA,
          pltpu.SemaphoreType.DMA,
      ],
  )
  def k(src_hbm, dst_hbm, p_hbm, q_hbm, out_hbm, sv, dv, bp, bq, sp, sq):
    c = lax.axis_index("c")
    s = lax.axis_index("s")
    my_chunks = jnp.where(c == 0, k0, k1)
    base_chunk = jnp.where(c == 0, s * k0, NS * k0 + s * k1)

    def chunk(j, _):
      off = (base_chunk + j) * CH
      pltpu.sync_copy(src_hbm.at[pl.ds(off, CH)], sv)
      pltpu.sync_copy(dst_hbm.at[pl.ds(off, CH)], dv)
      cp = pltpu.async_copy(p_hbm.at[sv], bp, sp)
      cq = pltpu.async_copy(q_hbm.at[dv], bq, sq)
      cp.wait()
      cq.wait()

      def addrow(r, _):
        for cc in range(h // LANES):
          sl = pl.ds(cc * LANES, LANES)
          bp[r, sl] = bp[r, sl] + bq[r, sl]
        return 0

      lax.fori_loop(0, CH, addrow, 0)
      pltpu.sync_copy(bp, out_hbm.at[pl.ds(off, CH)])
      return 0

    lax.fori_loop(0, my_chunks, chunk, 0)

  return k(src_pad, dst_pad, p_tab, q_tab)


def _tc_layer1(x_pad, w1, deg_t, bn):
  n_pad, d = x_pad.shape
  h = w1.shape[1]

  def body(x_ref, w_ref, deg_ref, g_ref, dinv_ref):
    deg = deg_ref[:, 0:1] + deg_ref[:, 1:2] + 1.0
    dinv = lax.rsqrt(deg)
    dinv_ref[...] = dinv
    g_ref[...] = jnp.dot(x_ref[...], w_ref[...],
                         preferred_element_type=jnp.float32) * dinv

  return pl.pallas_call(
      body,
      grid=(n_pad // bn,),
      in_specs=[
          pl.BlockSpec((bn, d), lambda i: (i, 0)),
          pl.BlockSpec((d, h), lambda i: (0, 0)),
          pl.BlockSpec((bn, 2), lambda i: (i, 0)),
      ],
      out_specs=[
          pl.BlockSpec((bn, h), lambda i: (i, 0)),
          pl.BlockSpec((bn, 1), lambda i: (i, 0)),
      ],
      out_shape=[
          jax.ShapeDtypeStruct((n_pad, h), jnp.float32),
          jax.ShapeDtypeStruct((n_pad, 1), jnp.float32),
      ],
  )(x_pad, w1, deg_t)


def _tc_layer2(g1, agg1, dinv, b1, w2, bn):
  n_pad, h = g1.shape
  d2 = w2.shape[1]

  def body(g_ref, agg_ref, dinv_ref, b_ref, w_ref, out_ref):
    dinv = dinv_ref[...]
    pre = (agg_ref[0] + agg_ref[1] + g_ref[...]) * dinv + b_ref[...]
    h1 = jnp.maximum(pre, 0.0)
    out_ref[...] = jnp.dot(h1, w_ref[...],
                           preferred_element_type=jnp.float32) * dinv

  return pl.pallas_call(
      body,
      grid=(n_pad // bn,),
      in_specs=[
          pl.BlockSpec((bn, h), lambda i: (i, 0)),
          pl.BlockSpec((NC, bn, h), lambda i: (0, i, 0)),
          pl.BlockSpec((bn, 1), lambda i: (i, 0)),
          pl.BlockSpec((1, h), lambda i: (0, 0)),
          pl.BlockSpec((h, d2), lambda i: (0, 0)),
      ],
      out_specs=pl.BlockSpec((bn, d2), lambda i: (i, 0)),
      out_shape=jax.ShapeDtypeStruct((n_pad, d2), jnp.float32),
  )(g1, agg1, dinv, b1, w2)


def _tc_node_mlp(g2, agg2, dinv, b2, wo1, bo1, wo2, bo2, we1a, we1b, be1, bn):
  n_pad, d = g2.shape
  h = wo1.shape[1]

  def body(g_ref, agg_ref, dinv_ref, b2_ref, wo1_ref, bo1_ref, wo2_ref,
           bo2_ref, wa_ref, wb_ref, be1_ref, h_ref, p_ref, q_ref):
    dinv = dinv_ref[...]
    pre = (agg_ref[0] + agg_ref[1] + g_ref[...]) * dinv + b2_ref[...]
    h2 = jnp.maximum(pre, 0.0)
    t = jnp.maximum(
        jnp.dot(h2, wo1_ref[...], preferred_element_type=jnp.float32)
        + bo1_ref[...], 0.0)
    hout = jnp.dot(t, wo2_ref[...],
                   preferred_element_type=jnp.float32) + bo2_ref[...]
    h_ref[...] = hout
    p_ref[...] = jnp.dot(hout, wa_ref[...],
                         preferred_element_type=jnp.float32) + be1_ref[...]
    q_ref[...] = jnp.dot(hout, wb_ref[...],
                         preferred_element_type=jnp.float32)

  return pl.pallas_call(
      body,
      grid=(n_pad // bn,),
      in_specs=[
          pl.BlockSpec((bn, d), lambda i: (i, 0)),
          pl.BlockSpec((NC, bn, d), lambda i: (0, i, 0)),
          pl.BlockSpec((bn, 1), lambda i: (i, 0)),
          pl.BlockSpec((1, d), lambda i: (0, 0)),
          pl.BlockSpec((d, h), lambda i: (0, 0)),
          pl.BlockSpec((1, h), lambda i: (0, 0)),
          pl.BlockSpec((h, d), lambda i: (0, 0)),
          pl.BlockSpec((1, d), lambda i: (0, 0)),
          pl.BlockSpec((d, h), lambda i: (0, 0)),
          pl.BlockSpec((d, h), lambda i: (0, 0)),
          pl.BlockSpec((1, h), lambda i: (0, 0)),
      ],
      out_specs=[
          pl.BlockSpec((bn, d), lambda i: (i, 0)),
          pl.BlockSpec((bn, h), lambda i: (i, 0)),
          pl.BlockSpec((bn, h), lambda i: (i, 0)),
      ],
      out_shape=[
          jax.ShapeDtypeStruct((n_pad, d), jnp.float32),
          jax.ShapeDtypeStruct((n_pad, h), jnp.float32),
          jax.ShapeDtypeStruct((n_pad, h), jnp.float32),
      ],
  )(g2, agg2, dinv, b2, wo1, bo1, wo2, bo2, we1a, we1b, be1)


def _tc_edge_mlp(s_arr, edge_attr, we1c, we2, be2, be):
  e_num, de = edge_attr.shape
  h = s_arr.shape[1]

  def body(s_ref, ea_ref, wc_ref, w2_ref, b2_ref, e_ref):
    t = jnp.maximum(
        s_ref[...] + jnp.dot(ea_ref[...], wc_ref[...],
                             preferred_element_type=jnp.float32), 0.0)
    e_ref[...] = jnp.dot(t, w2_ref[...],
                         preferred_element_type=jnp.float32) + b2_ref[...]

  return pl.pallas_call(
      body,
      grid=(e_num // be,),
      in_specs=[
          pl.BlockSpec((be, h), lambda i: (i, 0)),
          pl.BlockSpec((be, de), lambda i: (i, 0)),
          pl.BlockSpec((de, h), lambda i: (0, 0)),
          pl.BlockSpec((h, de), lambda i: (0, 0)),
          pl.BlockSpec((1, de), lambda i: (0, 0)),
      ],
      out_specs=pl.BlockSpec((be, de), lambda i: (i, 0)),
      out_shape=jax.ShapeDtypeStruct((e_num, de), jnp.float32),
  )(s_arr, edge_attr, we1c, we2, be2)


def kernel(x, edge_index, edge_attr, W1, b1, W2, b2, Wo1, bo1, Wo2, bo2,
           We1, be1, We2, be2):
  n, d = x.shape
  h = W1.shape[1]
  e_num = edge_index.shape[1]

  e_pad = _round_up(e_num, NW * CH)
  n_pad = _round_up(n + 1, NS * CH)

  src = edge_index[0]
  dst = edge_index[1]
  pad_idx = jnp.full((e_pad - e_num,), n, jnp.int32)
  src_pad = jnp.concatenate([src, pad_idx])
  dst_pad = jnp.concatenate([dst, pad_idx])
  x_pad = jnp.zeros((n_pad, d), jnp.float32).at[:n].set(x)

  bn = 1024
  assert n_pad % bn == 0

  degp = _sc_deg(e_pad, n_pad, dst_pad)
  deg_t = degp.T

  g1, dinv = _tc_layer1(x_pad, W1, deg_t, bn)
  total_per_sub = e_pad // CH // NS
  k0 = (total_per_sub * 13) // 20
  agg1 = _sc_aggregate(e_pad, n_pad, h, k0, src_pad, dst_pad, g1)
  g2 = _tc_layer2(g1, agg1, dinv, b1.reshape(1, -1), W2, bn)
  agg2 = _sc_aggregate(e_pad, n_pad, d, k0, src_pad, dst_pad, g2)
  hout, p_tab, q_tab = _tc_node_mlp(
      g2, agg2, dinv, b2.reshape(1, -1), Wo1, bo1.reshape(1, -1), Wo2,
      bo2.reshape(1, -1), We1[:d], We1[d:2 * d], be1.reshape(1, -1), bn)

  s_arr = _sc_pairgather(e_pad, h, k0, src_pad, dst_pad, p_tab, q_tab)
  e_out = _tc_edge_mlp(s_arr, edge_attr, We1[2 * d:], We2,
                       be2.reshape(1, -1), 2000)

  return hout[:n], e_out


# speedup vs baseline: 1.0842x; 1.0338x over previous
import functools

import jax
import jax.numpy as jnp
from jax import lax
from jax.experimental import pallas as pl
from jax.experimental.pallas import tpu as pltpu
from jax.experimental.pallas import tpu_sc as plsc

NC = 2
NS = 16
NW = NC * NS
CH = 128
LANES = 16


def _round_up(a, b):
  return (a + b - 1) // b * b


def _sc_deg(src_dst_len, n_pad, dst_pad):
  e_pad = src_dst_len
  per_w = e_pad // NW
  n_chunks = per_w // CH
  rows_per_tile = n_pad // NS
  mesh = plsc.VectorSubcoreMesh(core_axis_name="c", subcore_axis_name="s")

  @functools.partial(
      pl.kernel,
      out_type=jax.ShapeDtypeStruct((NC, n_pad), jnp.float32),
      mesh=mesh,
      scratch_types=[
          pltpu.VMEM((CH,), jnp.int32),
          pltpu.VMEM((CH,), jnp.float32),
          pltpu.VMEM((CH,), jnp.float32),
          pltpu.VMEM_SHARED((n_pad,), jnp.float32),
      ],
  )
  def k(dst_hbm, out_hbm, dv, ones_v, zbuf, deg_sh):
    c = lax.axis_index("c")
    s = lax.axis_index("s")
    wid = c * NS + s
    one16 = jnp.full((LANES,), 1.0, jnp.float32)
    z16 = jnp.zeros((LANES,), jnp.float32)
    for i in range(CH // LANES):
      ones_v[pl.ds(i * LANES, LANES)] = one16
      zbuf[pl.ds(i * LANES, LANES)] = z16
    tile_base = s * rows_per_tile
    for i in range(rows_per_tile // CH):
      pltpu.sync_copy(zbuf, deg_sh.at[pl.ds(tile_base + i * CH, CH)])
    plsc.subcore_barrier()

    base = wid * per_w

    def chunk(j, _):
      off = base + j * CH
      pltpu.sync_copy(dst_hbm.at[pl.ds(off, CH)], dv)
      pltpu.sync_copy(ones_v, deg_sh.at[dv], add=True)
      return 0

    lax.fori_loop(0, n_chunks, chunk, 0)
    plsc.subcore_barrier()
    for i in range(rows_per_tile // CH):
      r0 = tile_base + i * CH
      pltpu.sync_copy(deg_sh.at[pl.ds(r0, CH)], zbuf)
      pltpu.sync_copy(zbuf, out_hbm.at[c, pl.ds(r0, CH)])

  return k(dst_pad)


def _sc_aggregate(e_pad, n_pad, h, k0, idx_pairs, table):
  total_chunks = e_pad // CH
  k1 = total_chunks // NS - k0
  rows_per_tile = n_pad // NS
  mesh = plsc.VectorSubcoreMesh(core_axis_name="c", subcore_axis_name="s")

  @functools.partial(
      pl.kernel,
      out_type=jax.ShapeDtypeStruct((NC, n_pad, h), jnp.float32),
      mesh=mesh,
      scratch_types=[
          pltpu.VMEM((2, CH), jnp.int32),
          pltpu.VMEM((CH, h), jnp.float32),
          pltpu.VMEM_SHARED((n_pad, h), jnp.float32),
          pltpu.SemaphoreType.DMA,
      ],
  )
  def k(idx_hbm, table_hbm, out_hbm, iv, buf, acc_sh, gsem):
    c = lax.axis_index("c")
    s = lax.axis_index("s")
    wid = c * NS + s
    z16 = jnp.zeros((LANES,), jnp.float32)

    def zrow(r, _):
      for cc in range(h // LANES):
        buf[r, pl.ds(cc * LANES, LANES)] = z16
      return 0

    lax.fori_loop(0, CH, zrow, 0)
    tile_base = s * rows_per_tile
    for i in range(rows_per_tile // CH):
      pltpu.sync_copy(buf, acc_sh.at[pl.ds(tile_base + i * CH, CH)])
    plsc.subcore_barrier()

    my_chunks = jnp.where(c == 0, k0, k1)
    base_chunk = jnp.where(c == 0, s * k0, NS * k0 + s * k1)

    def chunk(j, _):
      pltpu.sync_copy(idx_hbm.at[base_chunk + j], iv)
      pltpu.async_copy(table_hbm.at[iv.at[0]], buf, gsem).wait()
      pltpu.sync_copy(buf, acc_sh.at[iv.at[1]], add=True)
      return 0

    lax.fori_loop(0, my_chunks, chunk, 0)
    plsc.subcore_barrier()
    for i in range(rows_per_tile // CH):
      r0 = tile_base + i * CH
      pltpu.sync_copy(acc_sh.at[pl.ds(r0, CH)], buf)
      pltpu.sync_copy(buf, out_hbm.at[c, pl.ds(r0, CH)])

  return k(idx_pairs, table)


def _sc_pairgather(e_pad, h, k0, idx_pairs, p_tab, q_tab):
  total_chunks = e_pad // CH
  k1 = total_chunks // NS - k0
  mesh = plsc.VectorSubcoreMesh(core_axis_name="c", subcore_axis_name="s")

  @functools.partial(
      pl.kernel,
      out_type=jax.ShapeDtypeStruct((e_pad, h), jnp.float32),
      mesh=mesh,
      scratch_types=[
          pltpu.VMEM((2, CH), jnp.int32),
          pltpu.VMEM((CH, h), jnp.float32),
          pltpu.VMEM((CH, h), jnp.float32),
          pltpu.SemaphoreType.DMA,
          pltpu.SemaphoreType.DMA,
      ],
  )
  def k(idx_hbm, p_hbm, q_hbm, out_hbm, iv, bp, bq, sp, sq):
    c = lax.axis_index("c")
    s = lax.axis_index("s")
    my_chunks = jnp.where(c == 0, k0, k1)
    base_chunk = jnp.where(c == 0, s * k0, NS * k0 + s * k1)

    def chunk(j, _):
      off = (base_chunk + j) * CH
      pltpu.sync_copy(idx_hbm.at[base_chunk + j], iv)
      cp = pltpu.async_copy(p_hbm.at[iv.at[0]], bp, sp)
      cq = pltpu.async_copy(q_hbm.at[iv.at[1]], bq, sq)
      cp.wait()
      cq.wait()

      def addrow(r, _):
        for cc in range(h // LANES):
          sl = pl.ds(cc * LANES, LANES)
          bp[r, sl] = bp[r, sl] + bq[r, sl]
        return 0

      lax.fori_loop(0, CH, addrow, 0)
      pltpu.sync_copy(bp, out_hbm.at[pl.ds(off, CH)])
      return 0

    lax.fori_loop(0, my_chunks, chunk, 0)

  return k(idx_pairs, p_tab, q_tab)


def _tc_layer1(x_pad, w1, deg_t, bn):
  n_pad, d = x_pad.shape
  h = w1.shape[1]

  def body(x_ref, w_ref, deg_ref, g_ref, dinv_ref):
    deg = deg_ref[:, 0:1] + deg_ref[:, 1:2] + 1.0
    dinv = lax.rsqrt(deg)
    dinv_ref[...] = dinv
    g_ref[...] = jnp.dot(x_ref[...], w_ref[...],
                         preferred_element_type=jnp.float32) * dinv

  return pl.pallas_call(
      body,
      grid=(n_pad // bn,),
      in_specs=[
          pl.BlockSpec((bn, d), lambda i: (i, 0)),
          pl.BlockSpec((d, h), lambda i: (0, 0)),
          pl.BlockSpec((bn, 2), lambda i: (i, 0)),
      ],
      out_specs=[
          pl.BlockSpec((bn, h), lambda i: (i, 0)),
          pl.BlockSpec((bn, 1), lambda i: (i, 0)),
      ],
      out_shape=[
          jax.ShapeDtypeStruct((n_pad, h), jnp.float32),
          jax.ShapeDtypeStruct((n_pad, 1), jnp.float32),
      ],
  )(x_pad, w1, deg_t)


def _tc_layer2(g1, agg1, dinv, b1, w2, bn):
  n_pad, h = g1.shape
  d2 = w2.shape[1]

  def body(g_ref, agg_ref, dinv_ref, b_ref, w_ref, out_ref):
    dinv = dinv_ref[...]
    pre = (agg_ref[0] + agg_ref[1] + g_ref[...]) * dinv + b_ref[...]
    h1 = jnp.maximum(pre, 0.0)
    out_ref[...] = jnp.dot(h1, w_ref[...],
                           preferred_element_type=jnp.float32) * dinv

  return pl.pallas_call(
      body,
      grid=(n_pad // bn,),
      in_specs=[
          pl.BlockSpec((bn, h), lambda i: (i, 0)),
          pl.BlockSpec((NC, bn, h), lambda i: (0, i, 0)),
          pl.BlockSpec((bn, 1), lambda i: (i, 0)),
          pl.BlockSpec((1, h), lambda i: (0, 0)),
          pl.BlockSpec((h, d2), lambda i: (0, 0)),
      ],
      out_specs=pl.BlockSpec((bn, d2), lambda i: (i, 0)),
      out_shape=jax.ShapeDtypeStruct((n_pad, d2), jnp.float32),
  )(g1, agg1, dinv, b1, w2)


def _tc_node_mlp(g2, agg2, dinv, b2, wo1, bo1, wo2, bo2, we1a, we1b, be1, bn):
  n_pad, d = g2.shape
  h = wo1.shape[1]

  def body(g_ref, agg_ref, dinv_ref, b2_ref, wo1_ref, bo1_ref, wo2_ref,
           bo2_ref, wa_ref, wb_ref, be1_ref, h_ref, p_ref, q_ref):
    dinv = dinv_ref[...]
    pre = (agg_ref[0] + agg_ref[1] + g_ref[...]) * dinv + b2_ref[...]
    h2 = jnp.maximum(pre, 0.0)
    t = jnp.maximum(
        jnp.dot(h2, wo1_ref[...], preferred_element_type=jnp.float32)
        + bo1_ref[...], 0.0)
    hout = jnp.dot(t, wo2_ref[...],
                   preferred_element_type=jnp.float32) + bo2_ref[...]
    h_ref[...] = hout
    p_ref[...] = jnp.dot(hout, wa_ref[...],
                         preferred_element_type=jnp.float32) + be1_ref[...]
    q_ref[...] = jnp.dot(hout, wb_ref[...],
                         preferred_element_type=jnp.float32)

  return pl.pallas_call(
      body,
      grid=(n_pad // bn,),
      in_specs=[
          pl.BlockSpec((bn, d), lambda i: (i, 0)),
          pl.BlockSpec((NC, bn, d), lambda i: (0, i, 0)),
          pl.BlockSpec((bn, 1), lambda i: (i, 0)),
          pl.BlockSpec((1, d), lambda i: (0, 0)),
          pl.BlockSpec((d, h), lambda i: (0, 0)),
          pl.BlockSpec((1, h), lambda i: (0, 0)),
          pl.BlockSpec((h, d), lambda i: (0, 0)),
          pl.BlockSpec((1, d), lambda i: (0, 0)),
          pl.BlockSpec((d, h), lambda i: (0, 0)),
          pl.BlockSpec((d, h), lambda i: (0, 0)),
          pl.BlockSpec((1, h), lambda i: (0, 0)),
      ],
      out_specs=[
          pl.BlockSpec((bn, d), lambda i: (i, 0)),
          pl.BlockSpec((bn, h), lambda i: (i, 0)),
          pl.BlockSpec((bn, h), lambda i: (i, 0)),
      ],
      out_shape=[
          jax.ShapeDtypeStruct((n_pad, d), jnp.float32),
          jax.ShapeDtypeStruct((n_pad, h), jnp.float32),
          jax.ShapeDtypeStruct((n_pad, h), jnp.float32),
      ],
  )(g2, agg2, dinv, b2, wo1, bo1, wo2, bo2, we1a, we1b, be1)


def _tc_edge_mlp(s_arr, edge_attr, we1c, we2, be2, be):
  e_num, de = edge_attr.shape
  h = s_arr.shape[1]

  def body(s_ref, ea_ref, wc_ref, w2_ref, b2_ref, e_ref):
    t = jnp.maximum(
        s_ref[...] + jnp.dot(ea_ref[...], wc_ref[...],
                             preferred_element_type=jnp.float32), 0.0)
    e_ref[...] = jnp.dot(t, w2_ref[...],
                         preferred_element_type=jnp.float32) + b2_ref[...]

  return pl.pallas_call(
      body,
      grid=(e_num // be,),
      in_specs=[
          pl.BlockSpec((be, h), lambda i: (i, 0)),
          pl.BlockSpec((be, de), lambda i: (i, 0)),
          pl.BlockSpec((de, h), lambda i: (0, 0)),
          pl.BlockSpec((h, de), lambda i: (0, 0)),
          pl.BlockSpec((1, de), lambda i: (0, 0)),
      ],
      out_specs=pl.BlockSpec((be, de), lambda i: (i, 0)),
      out_shape=jax.ShapeDtypeStruct((e_num, de), jnp.float32),
  )(s_arr, edge_attr, we1c, we2, be2)


def kernel(x, edge_index, edge_attr, W1, b1, W2, b2, Wo1, bo1, Wo2, bo2,
           We1, be1, We2, be2):
  n, d = x.shape
  h = W1.shape[1]
  e_num = edge_index.shape[1]

  e_pad = _round_up(e_num, NW * CH)
  n_pad = _round_up(n + 1, NS * CH)

  src = edge_index[0]
  dst = edge_index[1]
  pad_idx = jnp.full((e_pad - e_num,), n, jnp.int32)
  src_pad = jnp.concatenate([src, pad_idx])
  dst_pad = jnp.concatenate([dst, pad_idx])
  total_chunks = e_pad // CH
  idx_pairs = jnp.stack(
      [src_pad.reshape(total_chunks, CH), dst_pad.reshape(total_chunks, CH)],
      axis=1)
  x_pad = jnp.zeros((n_pad, d), jnp.float32).at[:n].set(x)

  bn = 1024
  assert n_pad % bn == 0

  degp = _sc_deg(e_pad, n_pad, dst_pad)
  deg_t = degp.T

  g1, dinv = _tc_layer1(x_pad, W1, deg_t, bn)
  total_per_sub = e_pad // CH // NS
  k0 = (total_per_sub * 13) // 20
  agg1 = _sc_aggregate(e_pad, n_pad, h, k0, idx_pairs, g1)
  g2 = _tc_layer2(g1, agg1, dinv, b1.reshape(1, -1), W2, bn)
  agg2 = _sc_aggregate(e_pad, n_pad, d, k0, idx_pairs, g2)
  hout, p_tab, q_tab = _tc_node_mlp(
      g2, agg2, dinv, b2.reshape(1, -1), Wo1, bo1.reshape(1, -1), Wo2,
      bo2.reshape(1, -1), We1[:d], We1[d:2 * d], be1.reshape(1, -1), bn)

  s_arr = _sc_pairgather(e_pad, h, k0, idx_pairs, p_tab, q_tab)
  e_out = _tc_edge_mlp(s_arr, edge_attr, We1[2 * d:], We2,
                       be2.reshape(1, -1), 2000)

  return hout[:n], e_out


# speedup vs baseline: 1.0951x; 1.0101x over previous
import functools

import jax
import jax.numpy as jnp
from jax import lax
from jax.experimental import pallas as pl
from jax.experimental.pallas import tpu as pltpu
from jax.experimental.pallas import tpu_sc as plsc

NC = 2
NS = 16
NW = NC * NS
CH = 128
LANES = 16


def _round_up(a, b):
  return (a + b - 1) // b * b


def _sc_deg(n_chunks, n_pad, dst3):
  rows_per_tile = n_pad // NS
  mesh = plsc.VectorSubcoreMesh(core_axis_name="c", subcore_axis_name="s")

  @functools.partial(
      pl.kernel,
      out_type=jax.ShapeDtypeStruct((NC, n_pad), jnp.float32),
      mesh=mesh,
      scratch_types=[
          pltpu.VMEM((n_chunks, CH), jnp.int32),
          pltpu.VMEM((CH,), jnp.float32),
          pltpu.VMEM((CH,), jnp.float32),
          pltpu.VMEM_SHARED((n_pad,), jnp.float32),
      ],
  )
  def k(dst_hbm, out_hbm, dv_all, ones_v, zbuf, deg_sh):
    c = lax.axis_index("c")
    s = lax.axis_index("s")
    wid = c * NS + s
    pltpu.sync_copy(dst_hbm.at[wid], dv_all)
    one16 = jnp.full((LANES,), 1.0, jnp.float32)
    z16 = jnp.zeros((LANES,), jnp.float32)
    for i in range(CH // LANES):
      ones_v[pl.ds(i * LANES, LANES)] = one16
      zbuf[pl.ds(i * LANES, LANES)] = z16
    tile_base = s * rows_per_tile
    for i in range(rows_per_tile // CH):
      pltpu.sync_copy(zbuf, deg_sh.at[pl.ds(tile_base + i * CH, CH)])
    plsc.subcore_barrier()

    def chunk(j, _):
      pltpu.sync_copy(ones_v, deg_sh.at[dv_all.at[j]], add=True)
      return 0

    lax.fori_loop(0, n_chunks, chunk, 0)
    plsc.subcore_barrier()
    for i in range(rows_per_tile // CH):
      r0 = tile_base + i * CH
      pltpu.sync_copy(deg_sh.at[pl.ds(r0, CH)], zbuf)
      pltpu.sync_copy(zbuf, out_hbm.at[c, pl.ds(r0, CH)])

  return k(dst3)


def _sc_aggregate(e_pad, n_pad, h, k0, idx_pairs, table):
  total_chunks = e_pad // CH
  k1 = total_chunks // NS - k0
  rows_per_tile = n_pad // NS
  mesh = plsc.VectorSubcoreMesh(core_axis_name="c", subcore_axis_name="s")

  @functools.partial(
      pl.kernel,
      out_type=jax.ShapeDtypeStruct((NC, n_pad, h), jnp.float32),
      mesh=mesh,
      scratch_types=[
          pltpu.VMEM((2, CH), jnp.int32),
          pltpu.VMEM((CH, h), jnp.float32),
          pltpu.VMEM_SHARED((n_pad, h), jnp.float32),
          pltpu.SemaphoreType.DMA,
      ],
  )
  def k(idx_hbm, table_hbm, out_hbm, iv, buf, acc_sh, gsem):
    c = lax.axis_index("c")
    s = lax.axis_index("s")
    wid = c * NS + s
    z16 = jnp.zeros((LANES,), jnp.float32)

    def zrow(r, _):
      for cc in range(h // LANES):
        buf[r, pl.ds(cc * LANES, LANES)] = z16
      return 0

    lax.fori_loop(0, CH, zrow, 0)
    tile_base = s * rows_per_tile
    for i in range(rows_per_tile // CH):
      pltpu.sync_copy(buf, acc_sh.at[pl.ds(tile_base + i * CH, CH)])
    plsc.subcore_barrier()

    my_chunks = jnp.where(c == 0, k0, k1)
    base_chunk = jnp.where(c == 0, s * k0, NS * k0 + s * k1)

    def chunk(j, _):
      pltpu.sync_copy(idx_hbm.at[base_chunk + j], iv)
      pltpu.async_copy(table_hbm.at[iv.at[0]], buf, gsem).wait()
      pltpu.sync_copy(buf, acc_sh.at[iv.at[1]], add=True)
      return 0

    lax.fori_loop(0, my_chunks, chunk, 0)
    plsc.subcore_barrier()
    for i in range(rows_per_tile // CH):
      r0 = tile_base + i * CH
      pltpu.sync_copy(acc_sh.at[pl.ds(r0, CH)], buf)
      pltpu.sync_copy(buf, out_hbm.at[c, pl.ds(r0, CH)])

  return k(idx_pairs, table)


def _sc_pairgather(e_pad, h, k0, idx_pairs, p_tab, q_tab):
  total_chunks = e_pad // CH
  k1 = total_chunks // NS - k0
  mesh = plsc.VectorSubcoreMesh(core_axis_name="c", subcore_axis_name="s")

  @functools.partial(
      pl.kernel,
      out_type=jax.ShapeDtypeStruct((e_pad, h), jnp.float32),
      mesh=mesh,
      scratch_types=[
          pltpu.VMEM((2, CH), jnp.int32),
          pltpu.VMEM((CH, h), jnp.float32),
          pltpu.VMEM((CH, h), jnp.float32),
          pltpu.SemaphoreType.DMA,
          pltpu.SemaphoreType.DMA,
      ],
  )
  def k(idx_hbm, p_hbm, q_hbm, out_hbm, iv, bp, bq, sp, sq):
    c = lax.axis_index("c")
    s = lax.axis_index("s")
    my_chunks = jnp.where(c == 0, k0, k1)
    base_chunk = jnp.where(c == 0, s * k0, NS * k0 + s * k1)

    def chunk(j, _):
      off = (base_chunk + j) * CH
      pltpu.sync_copy(idx_hbm.at[base_chunk + j], iv)
      cp = pltpu.async_copy(p_hbm.at[iv.at[0]], bp, sp)
      cq = pltpu.async_copy(q_hbm.at[iv.at[1]], bq, sq)
      cp.wait()
      cq.wait()

      def addrow(r, _):
        for cc in range(h // LANES):
          sl = pl.ds(cc * LANES, LANES)
          bp[r, sl] = bp[r, sl] + bq[r, sl]
        return 0

      lax.fori_loop(0, CH, addrow, 0)
      pltpu.sync_copy(bp, out_hbm.at[pl.ds(off, CH)])
      return 0

    lax.fori_loop(0, my_chunks, chunk, 0)

  return k(idx_pairs, p_tab, q_tab)


def _tc_layer1(x, n_pad, w1, deg_t, bn):
  d = x.shape[1]
  h = w1.shape[1]

  def body(x_ref, w_ref, deg_ref, g_ref, dinv_ref):
    deg = deg_ref[:, 0:1] + deg_ref[:, 1:2] + 1.0
    dinv = lax.rsqrt(deg)
    dinv_ref[...] = dinv
    g_ref[...] = jnp.dot(x_ref[...], w_ref[...],
                         preferred_element_type=jnp.float32) * dinv

  return pl.pallas_call(
      body,
      grid=(n_pad // bn,),
      in_specs=[
          pl.BlockSpec((bn, d), lambda i: (i, 0)),
          pl.BlockSpec((d, h), lambda i: (0, 0)),
          pl.BlockSpec((bn, 2), lambda i: (i, 0)),
      ],
      out_specs=[
          pl.BlockSpec((bn, h), lambda i: (i, 0)),
          pl.BlockSpec((bn, 1), lambda i: (i, 0)),
      ],
      out_shape=[
          jax.ShapeDtypeStruct((n_pad, h), jnp.float32),
          jax.ShapeDtypeStruct((n_pad, 1), jnp.float32),
      ],
  )(x, w1, deg_t)


def _tc_layer2(g1, agg1, dinv, b1, w2, bn):
  n_pad, h = g1.shape
  d2 = w2.shape[1]

  def body(g_ref, agg_ref, dinv_ref, b_ref, w_ref, out_ref):
    dinv = dinv_ref[...]
    pre = (agg_ref[0] + agg_ref[1] + g_ref[...]) * dinv + b_ref[...]
    h1 = jnp.maximum(pre, 0.0)
    out_ref[...] = jnp.dot(h1, w_ref[...],
                           preferred_element_type=jnp.float32) * dinv

  return pl.pallas_call(
      body,
      grid=(n_pad // bn,),
      in_specs=[
          pl.BlockSpec((bn, h), lambda i: (i, 0)),
          pl.BlockSpec((NC, bn, h), lambda i: (0, i, 0)),
          pl.BlockSpec((bn, 1), lambda i: (i, 0)),
          pl.BlockSpec((1, h), lambda i: (0, 0)),
          pl.BlockSpec((h, d2), lambda i: (0, 0)),
      ],
      out_specs=pl.BlockSpec((bn, d2), lambda i: (i, 0)),
      out_shape=jax.ShapeDtypeStruct((n_pad, d2), jnp.float32),
  )(g1, agg1, dinv, b1, w2)


def _tc_node_mlp(n, g2, agg2, dinv, b2, wo1, bo1, wo2, bo2, we1a, we1b, be1,
                 bn):
  n_pad, d = g2.shape
  h = wo1.shape[1]

  def body(g_ref, agg_ref, dinv_ref, b2_ref, wo1_ref, bo1_ref, wo2_ref,
           bo2_ref, wa_ref, wb_ref, be1_ref, h_ref, p_ref, q_ref):
    dinv = dinv_ref[...]
    pre = (agg_ref[0] + agg_ref[1] + g_ref[...]) * dinv + b2_ref[...]
    h2 = jnp.maximum(pre, 0.0)
    t = jnp.maximum(
        jnp.dot(h2, wo1_ref[...], preferred_element_type=jnp.float32)
        + bo1_ref[...], 0.0)
    hout = jnp.dot(t, wo2_ref[...],
                   preferred_element_type=jnp.float32) + bo2_ref[...]
    h_ref[...] = hout
    p_ref[...] = jnp.dot(hout, wa_ref[...],
                         preferred_element_type=jnp.float32) + be1_ref[...]
    q_ref[...] = jnp.dot(hout, wb_ref[...],
                         preferred_element_type=jnp.float32)

  return pl.pallas_call(
      body,
      grid=(n_pad // bn,),
      in_specs=[
          pl.BlockSpec((bn, d), lambda i: (i, 0)),
          pl.BlockSpec((NC, bn, d), lambda i: (0, i, 0)),
          pl.BlockSpec((bn, 1), lambda i: (i, 0)),
          pl.BlockSpec((1, d), lambda i: (0, 0)),
          pl.BlockSpec((d, h), lambda i: (0, 0)),
          pl.BlockSpec((1, h), lambda i: (0, 0)),
          pl.BlockSpec((h, d), lambda i: (0, 0)),
          pl.BlockSpec((1, d), lambda i: (0, 0)),
          pl.BlockSpec((d, h), lambda i: (0, 0)),
          pl.BlockSpec((d, h), lambda i: (0, 0)),
          pl.BlockSpec((1, h), lambda i: (0, 0)),
      ],
      out_specs=[
          pl.BlockSpec((bn, d), lambda i: (i, 0)),
          pl.BlockSpec((bn, h), lambda i: (i, 0)),
          pl.BlockSpec((bn, h), lambda i: (i, 0)),
      ],
      out_shape=[
          jax.ShapeDtypeStruct((n, d), jnp.float32),
          jax.ShapeDtypeStruct((n_pad, h), jnp.float32),
          jax.ShapeDtypeStruct((n_pad, h), jnp.float32),
      ],
  )(g2, agg2, dinv, b2, wo1, bo1, wo2, bo2, we1a, we1b, be1)


def _tc_edge_mlp(s_arr, edge_attr, we1c, we2, be2, be):
  e_num, de = edge_attr.shape
  h = s_arr.shape[1]

  def body(s_ref, ea_ref, wc_ref, w2_ref, b2_ref, e_ref):
    t = jnp.maximum(
        s_ref[...] + jnp.dot(ea_ref[...], wc_ref[...],
                             preferred_element_type=jnp.float32), 0.0)
    e_ref[...] = jnp.dot(t, w2_ref[...],
                         preferred_element_type=jnp.float32) + b2_ref[...]

  return pl.pallas_call(
      body,
      grid=(e_num // be,),
      in_specs=[
          pl.BlockSpec((be, h), lambda i: (i, 0)),
          pl.BlockSpec((be, de), lambda i: (i, 0)),
          pl.BlockSpec((de, h), lambda i: (0, 0)),
          pl.BlockSpec((h, de), lambda i: (0, 0)),
          pl.BlockSpec((1, de), lambda i: (0, 0)),
      ],
      out_specs=pl.BlockSpec((be, de), lambda i: (i, 0)),
      out_shape=jax.ShapeDtypeStruct((e_num, de), jnp.float32),
  )(s_arr, edge_attr, we1c, we2, be2)


def kernel(x, edge_index, edge_attr, W1, b1, W2, b2, Wo1, bo1, Wo2, bo2,
           We1, be1, We2, be2):
  n, d = x.shape
  h = W1.shape[1]
  e_num = edge_index.shape[1]

  e_pad = _round_up(e_num, NW * CH)
  n_pad = _round_up(n + 1, NS * CH)

  src = edge_index[0]
  dst = edge_index[1]
  pad_idx = jnp.full((e_pad - e_num,), n, jnp.int32)
  src_pad = jnp.concatenate([src, pad_idx])
  dst_pad = jnp.concatenate([dst, pad_idx])
  total_chunks = e_pad // CH
  idx_pairs = jnp.stack(
      [src_pad.reshape(total_chunks, CH), dst_pad.reshape(total_chunks, CH)],
      axis=1)

  bn = 1024
  assert n_pad % bn == 0

  degp = _sc_deg(e_pad // (NW * CH), n_pad, dst_pad.reshape(NW, e_pad // (NW * CH), CH))
  deg_t = degp.T

  g1, dinv = _tc_layer1(x, n_pad, W1, deg_t, bn)
  total_per_sub = e_pad // CH // NS
  k0 = (total_per_sub * 13) // 20
  agg1 = _sc_aggregate(e_pad, n_pad, h, k0, idx_pairs, g1)
  g2 = _tc_layer2(g1, agg1, dinv, b1.reshape(1, -1), W2, bn)
  agg2 = _sc_aggregate(e_pad, n_pad, d, k0, idx_pairs, g2)
  hout, p_tab, q_tab = _tc_node_mlp(
      n, g2, agg2, dinv, b2.reshape(1, -1), Wo1, bo1.reshape(1, -1), Wo2,
      bo2.reshape(1, -1), We1[:d], We1[d:2 * d], be1.reshape(1, -1), bn)

  s_arr = _sc_pairgather(e_pad, h, k0, idx_pairs, p_tab, q_tab)
  e_out = _tc_edge_mlp(s_arr, edge_attr, We1[2 * d:], We2,
                       be2.reshape(1, -1), 2000)

  return hout, e_out


# speedup vs baseline: 1.1047x; 1.0087x over previous
import functools

import jax
import jax.numpy as jnp
from jax import lax
from jax.experimental import pallas as pl
from jax.experimental.pallas import tpu as pltpu
from jax.experimental.pallas import tpu_sc as plsc

NC = 2
NS = 16
NW = NC * NS
CH = 128
LANES = 16


def _round_up(a, b):
  return (a + b - 1) // b * b


def _sc_deg(n_chunks, n_pad, dst3):
  rows_per_tile = n_pad // NS
  mesh = plsc.VectorSubcoreMesh(core_axis_name="c", subcore_axis_name="s")

  @functools.partial(
      pl.kernel,
      out_type=jax.ShapeDtypeStruct((NC, n_pad), jnp.float32),
      mesh=mesh,
      scratch_types=[
          pltpu.VMEM((n_chunks, CH), jnp.int32),
          pltpu.VMEM((CH,), jnp.float32),
          pltpu.VMEM((CH,), jnp.float32),
          pltpu.VMEM_SHARED((n_pad,), jnp.float32),
      ],
  )
  def k(dst_hbm, out_hbm, dv_all, ones_v, zbuf, deg_sh):
    c = lax.axis_index("c")
    s = lax.axis_index("s")
    wid = c * NS + s
    pltpu.sync_copy(dst_hbm.at[wid], dv_all)
    one16 = jnp.full((LANES,), 1.0, jnp.float32)
    z16 = jnp.zeros((LANES,), jnp.float32)
    for i in range(CH // LANES):
      ones_v[pl.ds(i * LANES, LANES)] = one16
      zbuf[pl.ds(i * LANES, LANES)] = z16
    tile_base = s * rows_per_tile
    for i in range(rows_per_tile // CH):
      pltpu.sync_copy(zbuf, deg_sh.at[pl.ds(tile_base + i * CH, CH)])
    plsc.subcore_barrier()

    def chunk(j, _):
      pltpu.sync_copy(ones_v, deg_sh.at[dv_all.at[j]], add=True)
      return 0

    lax.fori_loop(0, n_chunks, chunk, 0)
    plsc.subcore_barrier()
    for i in range(rows_per_tile // CH):
      r0 = tile_base + i * CH
      pltpu.sync_copy(deg_sh.at[pl.ds(r0, CH)], zbuf)
      pltpu.sync_copy(zbuf, out_hbm.at[c, pl.ds(r0, CH)])

  return k(dst3)


def _sc_aggregate(e_pad, n_pad, h, k0, idx_pairs, table):
  total_chunks = e_pad // CH
  k1 = total_chunks // NS - k0
  rows_per_tile = n_pad // NS
  mesh = plsc.VectorSubcoreMesh(core_axis_name="c", subcore_axis_name="s")

  @functools.partial(
      pl.kernel,
      out_type=jax.ShapeDtypeStruct((NC, n_pad, h), jnp.float32),
      mesh=mesh,
      scratch_types=[
          pltpu.VMEM((2, CH), jnp.int32),
          pltpu.VMEM((CH, h), jnp.float32),
          pltpu.VMEM_SHARED((n_pad, h), jnp.float32),
          pltpu.SemaphoreType.DMA,
      ],
  )
  def k(idx_hbm, table_hbm, out_hbm, iv, buf, acc_sh, gsem):
    c = lax.axis_index("c")
    s = lax.axis_index("s")
    wid = c * NS + s
    z16 = jnp.zeros((LANES,), jnp.float32)

    def zrow(r, _):
      for cc in range(h // LANES):
        buf[r, pl.ds(cc * LANES, LANES)] = z16
      return 0

    lax.fori_loop(0, CH, zrow, 0)
    tile_base = s * rows_per_tile
    for i in range(rows_per_tile // CH):
      pltpu.sync_copy(buf, acc_sh.at[pl.ds(tile_base + i * CH, CH)])
    plsc.subcore_barrier()

    my_chunks = jnp.where(c == 0, k0, k1)
    base_chunk = jnp.where(c == 0, s * k0, NS * k0 + s * k1)

    def chunk(j, _):
      pltpu.sync_copy(idx_hbm.at[base_chunk + j], iv)
      pltpu.async_copy(table_hbm.at[iv.at[0]], buf, gsem).wait()
      pltpu.sync_copy(buf, acc_sh.at[iv.at[1]], add=True)
      return 0

    lax.fori_loop(0, my_chunks, chunk, 0)
    plsc.subcore_barrier()
    for i in range(rows_per_tile // CH):
      r0 = tile_base + i * CH
      pltpu.sync_copy(acc_sh.at[pl.ds(r0, CH)], buf)
      pltpu.sync_copy(buf, out_hbm.at[c, pl.ds(r0, CH)])

  return k(idx_pairs, table)


def _sc_pairgather(e_pad, h, k0, idx_pairs, p_tab, q_tab):
  total_chunks = e_pad // CH
  k1 = total_chunks // NS - k0
  mesh = plsc.VectorSubcoreMesh(core_axis_name="c", subcore_axis_name="s")

  @functools.partial(
      pl.kernel,
      out_type=jax.ShapeDtypeStruct((e_pad, h), jnp.float32),
      mesh=mesh,
      scratch_types=[
          pltpu.VMEM((2, CH), jnp.int32),
          pltpu.VMEM((CH, h), jnp.float32),
          pltpu.VMEM((CH, h), jnp.float32),
          pltpu.SemaphoreType.DMA,
          pltpu.SemaphoreType.DMA,
      ],
  )
  def k(idx_hbm, p_hbm, q_hbm, out_hbm, iv, bp, bq, sp, sq):
    c = lax.axis_index("c")
    s = lax.axis_index("s")
    my_chunks = jnp.where(c == 0, k0, k1)
    base_chunk = jnp.where(c == 0, s * k0, NS * k0 + s * k1)

    def chunk(j, _):
      off = (base_chunk + j) * CH
      pltpu.sync_copy(idx_hbm.at[base_chunk + j], iv)
      cp = pltpu.async_copy(p_hbm.at[iv.at[0]], bp, sp)
      cq = pltpu.async_copy(q_hbm.at[iv.at[1]], bq, sq)
      cp.wait()
      cq.wait()

      def addrow(r, _):
        for cc in range(h // LANES):
          sl = pl.ds(cc * LANES, LANES)
          bp[r, sl] = bp[r, sl] + bq[r, sl]
        return 0

      lax.fori_loop(0, CH, addrow, 0)
      pltpu.sync_copy(bp, out_hbm.at[pl.ds(off, CH)])
      return 0

    lax.fori_loop(0, my_chunks, chunk, 0)

  return k(idx_pairs, p_tab, q_tab)


def _tc_matmul1(x, n_pad, w1, bn):
  d = x.shape[1]
  h = w1.shape[1]

  def body(x_ref, w_ref, g_ref):
    g_ref[...] = jnp.dot(x_ref[...], w_ref[...],
                         preferred_element_type=jnp.float32)

  return pl.pallas_call(
      body,
      grid=(n_pad // bn,),
      in_specs=[
          pl.BlockSpec((bn, d), lambda i: (i, 0)),
          pl.BlockSpec((d, h), lambda i: (0, 0)),
      ],
      out_specs=pl.BlockSpec((bn, h), lambda i: (i, 0)),
      out_shape=jax.ShapeDtypeStruct((n_pad, h), jnp.float32),
  )(x, w1)


def _tc_scale1(h1, deg_t, bn):
  n_pad, h = h1.shape

  def body(h_ref, deg_ref, g_ref, dinv_ref):
    deg = deg_ref[:, 0:1] + deg_ref[:, 1:2] + 1.0
    dinv = lax.rsqrt(deg)
    dinv_ref[...] = dinv
    g_ref[...] = h_ref[...] * dinv

  return pl.pallas_call(
      body,
      grid=(n_pad // bn,),
      in_specs=[
          pl.BlockSpec((bn, h), lambda i: (i, 0)),
          pl.BlockSpec((bn, 2), lambda i: (i, 0)),
      ],
      out_specs=[
          pl.BlockSpec((bn, h), lambda i: (i, 0)),
          pl.BlockSpec((bn, 1), lambda i: (i, 0)),
      ],
      out_shape=[
          jax.ShapeDtypeStruct((n_pad, h), jnp.float32),
          jax.ShapeDtypeStruct((n_pad, 1), jnp.float32),
      ],
  )(h1, deg_t)


def _tc_layer2(g1, agg1, dinv, b1, w2, bn):
  n_pad, h = g1.shape
  d2 = w2.shape[1]

  def body(g_ref, agg_ref, dinv_ref, b_ref, w_ref, out_ref):
    dinv = dinv_ref[...]
    pre = (agg_ref[0] + agg_ref[1] + g_ref[...]) * dinv + b_ref[...]
    h1 = jnp.maximum(pre, 0.0)
    out_ref[...] = jnp.dot(h1, w_ref[...],
                           preferred_element_type=jnp.float32) * dinv

  return pl.pallas_call(
      body,
      grid=(n_pad // bn,),
      in_specs=[
          pl.BlockSpec((bn, h), lambda i: (i, 0)),
          pl.BlockSpec((NC, bn, h), lambda i: (0, i, 0)),
          pl.BlockSpec((bn, 1), lambda i: (i, 0)),
          pl.BlockSpec((1, h), lambda i: (0, 0)),
          pl.BlockSpec((h, d2), lambda i: (0, 0)),
      ],
      out_specs=pl.BlockSpec((bn, d2), lambda i: (i, 0)),
      out_shape=jax.ShapeDtypeStruct((n_pad, d2), jnp.float32),
  )(g1, agg1, dinv, b1, w2)


def _tc_node_mlp(n, g2, agg2, dinv, b2, wo1, bo1, wo2, bo2, we1a, we1b, be1,
                 bn):
  n_pad, d = g2.shape
  h = wo1.shape[1]

  def body(g_ref, agg_ref, dinv_ref, b2_ref, wo1_ref, bo1_ref, wo2_ref,
           bo2_ref, wa_ref, wb_ref, be1_ref, h_ref, p_ref, q_ref):
    dinv = dinv_ref[...]
    pre = (agg_ref[0] + agg_ref[1] + g_ref[...]) * dinv + b2_ref[...]
    h2 = jnp.maximum(pre, 0.0)
    t = jnp.maximum(
        jnp.dot(h2, wo1_ref[...], preferred_element_type=jnp.float32)
        + bo1_ref[...], 0.0)
    hout = jnp.dot(t, wo2_ref[...],
                   preferred_element_type=jnp.float32) + bo2_ref[...]
    h_ref[...] = hout
    p_ref[...] = jnp.dot(hout, wa_ref[...],
                         preferred_element_type=jnp.float32) + be1_ref[...]
    q_ref[...] = jnp.dot(hout, wb_ref[...],
                         preferred_element_type=jnp.float32)

  return pl.pallas_call(
      body,
      grid=(n_pad // bn,),
      in_specs=[
          pl.BlockSpec((bn, d), lambda i: (i, 0)),
          pl.BlockSpec((NC, bn, d), lambda i: (0, i, 0)),
          pl.BlockSpec((bn, 1), lambda i: (i, 0)),
          pl.BlockSpec((1, d), lambda i: (0, 0)),
          pl.BlockSpec((d, h), lambda i: (0, 0)),
          pl.BlockSpec((1, h), lambda i: (0, 0)),
          pl.BlockSpec((h, d), lambda i: (0, 0)),
          pl.BlockSpec((1, d), lambda i: (0, 0)),
          pl.BlockSpec((d, h), lambda i: (0, 0)),
          pl.BlockSpec((d, h), lambda i: (0, 0)),
          pl.BlockSpec((1, h), lambda i: (0, 0)),
      ],
      out_specs=[
          pl.BlockSpec((bn, d), lambda i: (i, 0)),
          pl.BlockSpec((bn, h), lambda i: (i, 0)),
          pl.BlockSpec((bn, h), lambda i: (i, 0)),
      ],
      out_shape=[
          jax.ShapeDtypeStruct((n, d), jnp.float32),
          jax.ShapeDtypeStruct((n_pad, h), jnp.float32),
          jax.ShapeDtypeStruct((n_pad, h), jnp.float32),
      ],
  )(g2, agg2, dinv, b2, wo1, bo1, wo2, bo2, we1a, we1b, be1)


def _tc_edge_mlp(s_arr, edge_attr, we1c, we2, be2, be):
  e_num, de = edge_attr.shape
  h = s_arr.shape[1]

  def body(s_ref, ea_ref, wc_ref, w2_ref, b2_ref, e_ref):
    t = jnp.maximum(
        s_ref[...] + jnp.dot(ea_ref[...], wc_ref[...],
                             preferred_element_type=jnp.float32), 0.0)
    e_ref[...] = jnp.dot(t, w2_ref[...],
                         preferred_element_type=jnp.float32) + b2_ref[...]

  return pl.pallas_call(
      body,
      grid=(e_num // be,),
      in_specs=[
          pl.BlockSpec((be, h), lambda i: (i, 0)),
          pl.BlockSpec((be, de), lambda i: (i, 0)),
          pl.BlockSpec((de, h), lambda i: (0, 0)),
          pl.BlockSpec((h, de), lambda i: (0, 0)),
          pl.BlockSpec((1, de), lambda i: (0, 0)),
      ],
      out_specs=pl.BlockSpec((be, de), lambda i: (i, 0)),
      out_shape=jax.ShapeDtypeStruct((e_num, de), jnp.float32),
  )(s_arr, edge_attr, we1c, we2, be2)


def kernel(x, edge_index, edge_attr, W1, b1, W2, b2, Wo1, bo1, Wo2, bo2,
           We1, be1, We2, be2):
  n, d = x.shape
  h = W1.shape[1]
  e_num = edge_index.shape[1]

  e_pad = _round_up(e_num, NW * CH)
  n_pad = _round_up(n + 1, NS * CH)

  src = edge_index[0]
  dst = edge_index[1]
  pad_idx = jnp.full((e_pad - e_num,), n, jnp.int32)
  src_pad = jnp.concatenate([src, pad_idx])
  dst_pad = jnp.concatenate([dst, pad_idx])
  total_chunks = e_pad // CH
  idx_pairs = jnp.stack(
      [src_pad.reshape(total_chunks, CH), dst_pad.reshape(total_chunks, CH)],
      axis=1)

  bn = 1024
  assert n_pad % bn == 0

  degp = _sc_deg(e_pad // (NW * CH), n_pad, dst_pad.reshape(NW, e_pad // (NW * CH), CH))
  deg_t = degp.T

  h1 = _tc_matmul1(x, n_pad, W1, bn)
  g1, dinv = _tc_scale1(h1, deg_t, bn)
  total_per_sub = e_pad // CH // NS
  k0 = (total_per_sub * 13) // 20
  agg1 = _sc_aggregate(e_pad, n_pad, h, k0, idx_pairs, g1)
  g2 = _tc_layer2(g1, agg1, dinv, b1.reshape(1, -1), W2, bn)
  agg2 = _sc_aggregate(e_pad, n_pad, d, k0, idx_pairs, g2)
  hout, p_tab, q_tab = _tc_node_mlp(
      n, g2, agg2, dinv, b2.reshape(1, -1), Wo1, bo1.reshape(1, -1), Wo2,
      bo2.reshape(1, -1), We1[:d], We1[d:2 * d], be1.reshape(1, -1), bn)

  s_arr = _sc_pairgather(e_pad, h, k0, idx_pairs, p_tab, q_tab)
  e_out = _tc_edge_mlp(s_arr, edge_attr, We1[2 * d:], We2,
                       be2.reshape(1, -1), 2000)

  return hout, e_out
